# Initial kernel scaffold; baseline (speedup 1.0000x reference)
import jax
import jax.numpy as jnp
from jax.experimental import pallas as pl


def kernel(u, v, boundary_norm, is_boundary, y_force, pos, edge_index, params):
    raise NotImplementedError("write your pallas kernel here")


# speedup vs baseline: 2.3026x; 2.3026x over previous
import functools

import jax
import jax.numpy as jnp
from jax import lax
from jax.experimental import pallas as pl
from jax.experimental.pallas import tpu as pltpu
import jax.experimental.pallas.tpu_sc as plsc

_N = 50000
_E = 800000
_T = 10
_RDIM = 16
_NS = 32
_L = 5

_NP = 50176
_EP = 819200
_NWORK = 32
_PW = _EP // _NWORK
_CH = 128
_IB = 1024
_SUBROWS = _NP // 16

_BRN = 6272
_BRE = 8192

_f32 = jnp.float32
_i32 = jnp.int32


def _sds(shape):
    return jax.ShapeDtypeStruct(shape, _f32)


def _k_emb(u_ref, vx_ref, vy_ref, bn_ref, y_ref, w0_ref, b0_ref, w1_ref,
           b1_ref, o_ref):
    u = u_ref[...]
    vx = vx_ref[...]
    vy = vy_ref[...]
    bn = bn_ref[...]
    y = y_ref[...]
    vn = jnp.sqrt(vx * vx + vy * vy)
    bnx = bn[:, 0:1]
    bny = bn[:, 1:2]
    vd = vx * bnx + vy * bny
    bnn = jnp.sqrt(bnx * bnx + bny * bny)
    inv = jnp.concatenate([u, vn, vd, bnn, y], axis=1)
    h = jnp.dot(inv, w0_ref[...], preferred_element_type=_f32) + b0_ref[0:1, :]
    h = jnp.maximum(h, 0.0)
    o_ref[...] = jnp.dot(h, w1_ref[...], preferred_element_type=_f32) + b1_ref[0:1, :]


def _k_bessel(pr_ref, pc_ref, o_ref):
    dx = pr_ref[:, 0:1] - pc_ref[:, 0:1]
    dy = pr_ref[:, 1:2] - pc_ref[:, 1:2]
    d = jnp.sqrt(dx * dx + dy * dy)
    n = lax.broadcasted_iota(_i32, (1, _RDIM), 1).astype(_f32) + 1.0
    o_ref[...] = jnp.sqrt(2.0) * jnp.sin(n * jnp.pi * d) / (d + 1e-8)


def _k_ab(x_ref, wa_ref, wb_ref, b_ref, a_ref, bo_ref):
    x = x_ref[...]
    a_ref[...] = jnp.dot(x, wa_ref[...], preferred_element_type=_f32) + b_ref[0:1, :]
    bo_ref[...] = jnp.dot(x, wb_ref[...], preferred_element_type=_f32)


def _k_edge(gr_ref, gc_ref, de_ref, wc_ref, w2_ref, b2_ref, o_ref):
    pre = gr_ref[...] + gc_ref[...] + jnp.dot(
        de_ref[...], wc_ref[...], preferred_element_type=_f32)
    h = jnp.maximum(pre, 0.0)
    o_ref[...] = jnp.dot(h, w2_ref[...], preferred_element_type=_f32) + b2_ref[0:1, :]


def _k_node(x_ref, acc_ref, va_ref, vb_ref, c_ref, v2_ref, o_ref):
    x = x_ref[...]
    agg = acc_ref[0] + acc_ref[1]
    h = (jnp.dot(x, va_ref[...], preferred_element_type=_f32)
         + jnp.dot(agg, vb_ref[...], preferred_element_type=_f32)
         + c_ref[0:1, :])
    h = jnp.maximum(h, 0.0)
    o_ref[...] = x + jnp.dot(h, v2_ref[...], preferred_element_type=_f32) + c_ref[1:2, :]


def _k_out(x_ref, res_ref, ln_ref, w1_ref, b1_ref, w2_ref, b2_ref, o_ref):
    x = x_ref[...]
    mu = jnp.mean(x, axis=1, keepdims=True)
    xc = x - mu
    var = jnp.mean(xc * xc, axis=1, keepdims=True)
    xn = xc * jax.lax.rsqrt(var + 1e-5) * ln_ref[0:1, :] + ln_ref[1:2, :]
    h = jnp.dot(xn, w1_ref[...], preferred_element_type=_f32) + b1_ref[0:1, :]
    h = jnp.maximum(h, 0.0)
    o = jnp.dot(h, w2_ref[...], preferred_element_type=_f32) + b2_ref[0:1, :]
    o_ref[...] = o + res_ref[...]


def _tc_call(body, grid, in_specs, out_specs, out_shape):
    return pl.pallas_call(body, grid=grid, in_specs=in_specs,
                          out_specs=out_specs, out_shape=out_shape)


def _row_spec(br, w):
    return pl.BlockSpec((br, w), lambda i: (i, 0))


def _full_spec(shape):
    return pl.BlockSpec(shape, lambda i: tuple(0 for _ in shape))


@functools.cache
def _mesh():
    return plsc.VectorSubcoreMesh(core_axis_name="c", subcore_axis_name="s")


@functools.cache
def _make_gather(width):
    ngrp = _PW // _IB
    nchk = _IB // _CH

    def body(tab_a, tab_b, idx_a, idx_b, out_a, out_b, ia, ib, bufa, bufb,
             sa, sb):
        wid = lax.axis_index("s") * 2 + lax.axis_index("c")
        base = wid * _PW

        def group(g, carry):
            goff = base + g * _IB
            pltpu.sync_copy(idx_a.at[pl.ds(goff, _IB)], ia)
            pltpu.sync_copy(idx_b.at[pl.ds(goff, _IB)], ib)

            def chunk(j, cc):
                coff = goff + j * _CH
                ca = pltpu.async_copy(tab_a.at[ia.at[pl.ds(j * _CH, _CH)]],
                                      bufa, sa)
                cb = pltpu.async_copy(tab_b.at[ib.at[pl.ds(j * _CH, _CH)]],
                                      bufb, sb)
                ca.wait()
                cb.wait()
                pltpu.sync_copy(bufa, out_a.at[pl.ds(coff, _CH)])
                pltpu.sync_copy(bufb, out_b.at[pl.ds(coff, _CH)])
                return cc

            return lax.fori_loop(0, nchk, chunk, carry)

        lax.fori_loop(0, ngrp, group, 0)

    return pl.kernel(
        body,
        out_type=(_sds((_EP, width)), _sds((_EP, width))),
        mesh=_mesh(),
        scratch_types=[
            pltpu.VMEM((_IB,), _i32),
            pltpu.VMEM((_IB,), _i32),
            pltpu.VMEM((_CH, width), _f32),
            pltpu.VMEM((_CH, width), _f32),
            pltpu.SemaphoreType.DMA,
            pltpu.SemaphoreType.DMA,
        ],
        compiler_params=pltpu.CompilerParams(use_tc_tiling_on_sc=False),
    )


@functools.cache
def _make_scatter():
    zr = 392
    grp = 4
    ngrp = _PW // (grp * _CH)

    def body(m_hbm, row2d, out_hbm, idx2d, mbuf, zbuf, acc_sh):
        cid = lax.axis_index("c")
        sid = lax.axis_index("s")
        wid = sid * 2 + cid
        z = jnp.zeros((16,), _f32)

        def zb(r, carry):
            zbuf[r, pl.ds(0, 16)] = z
            zbuf[r, pl.ds(16, 16)] = z
            return carry

        lax.fori_loop(0, zr, zb, 0)

        def zs(i, carry):
            pltpu.sync_copy(zbuf, acc_sh.at[pl.ds(sid * _SUBROWS + i * zr, zr)])
            return carry

        lax.fori_loop(0, _SUBROWS // zr, zs, 0)
        plsc.subcore_barrier()

        cbase = wid * (_PW // _CH)

        def group(g, carry):
            c0 = cbase + g * grp
            pltpu.sync_copy(row2d.at[pl.ds(c0, grp)], idx2d)
            pltpu.sync_copy(m_hbm.at[pl.ds(c0 * _CH, grp * _CH)], mbuf)

            def chunk(j, cc):
                pltpu.sync_copy(mbuf.at[pl.ds(j * _CH, _CH)],
                                acc_sh.at[idx2d.at[j]], add=True)
                return cc

            return lax.fori_loop(0, grp, chunk, carry)

        lax.fori_loop(0, ngrp, group, 0)
        plsc.subcore_barrier()

        def dump(i, carry):
            r0 = sid * _SUBROWS + i * zr
            pltpu.sync_copy(acc_sh.at[pl.ds(r0, zr)], zbuf)
            pltpu.sync_copy(zbuf, out_hbm.at[cid, pl.ds(r0, zr)])
            return carry

        lax.fori_loop(0, _SUBROWS // zr, dump, 0)

    return pl.kernel(
        body,
        out_type=_sds((2, _NP, 32)),
        mesh=_mesh(),
        scratch_types=[
            pltpu.VMEM((grp, _CH), _i32),
            pltpu.VMEM((grp * _CH, 32), _f32),
            pltpu.VMEM((zr, 32), _f32),
            pltpu.VMEM_SHARED((_NP, 32), _f32),
        ],
        compiler_params=pltpu.CompilerParams(use_tc_tiling_on_sc=False),
    )


def _brow(vec, rows=8):
    return jnp.zeros((rows, vec.shape[0]), _f32).at[0].set(vec)


def _emb_call(u_p, vx, vy, bn_p, y_p, w0, b0, w1, b1):
    grid = (_NP // _BRN,)
    return _tc_call(
        _k_emb, grid,
        [
            _row_spec(_BRN, _T), _row_spec(_BRN, _T), _row_spec(_BRN, _T),
            _row_spec(_BRN, 2), _row_spec(_BRN, 1),
            _full_spec(w0.shape), _full_spec((8, 2 * _NS)),
            _full_spec(w1.shape), _full_spec((8, _NS)),
        ],
        _row_spec(_BRN, _NS), _sds((_NP, _NS)),
    )(u_p, vx, vy, bn_p, y_p, w0, _brow(b0), w1, _brow(b1))


def _bessel_call(pr, pc):
    grid = (_EP // _BRE,)
    return _tc_call(
        _k_bessel, grid,
        [_row_spec(_BRE, 16), _row_spec(_BRE, 16)],
        _row_spec(_BRE, _RDIM), _sds((_EP, _RDIM)),
    )(pr, pc)


def _ab_call(x, wa, wb, b1):
    grid = (_NP // _BRN,)
    return _tc_call(
        _k_ab, grid,
        [_row_spec(_BRN, _NS), _full_spec(wa.shape), _full_spec(wb.shape),
         _full_spec((8, _NS))],
        (_row_spec(_BRN, _NS), _row_spec(_BRN, _NS)),
        (_sds((_NP, _NS)), _sds((_NP, _NS))),
    )(x, wa, wb, _brow(b1))


def _edge_call(gr, gc, demb, wc, w2, b2):
    grid = (_EP // _BRE,)
    return _tc_call(
        _k_edge, grid,
        [_row_spec(_BRE, _NS), _row_spec(_BRE, _NS), _row_spec(_BRE, _RDIM),
         _full_spec(wc.shape), _full_spec(w2.shape), _full_spec((8, _NS))],
        _row_spec(_BRE, _NS), _sds((_EP, _NS)),
    )(gr, gc, demb, wc, w2, _brow(b2))


def _node_call(x, acc, va, vb, c1, v2, c2):
    grid = (_NP // _BRN,)
    crows = jnp.zeros((8, _NS), _f32).at[0].set(c1).at[1].set(c2)
    return _tc_call(
        _k_node, grid,
        [_row_spec(_BRN, _NS),
         pl.BlockSpec((2, _BRN, _NS), lambda i: (0, i, 0)),
         _full_spec(va.shape), _full_spec(vb.shape), _full_spec((8, _NS)),
         _full_spec(v2.shape)],
        _row_spec(_BRN, _NS), _sds((_NP, _NS)),
    )(x, acc, va, vb, crows, v2)


def _out_call(x, res, g, b, w1, b1, w2, b2):
    grid = (_NP // _BRN,)
    lnrows = jnp.zeros((8, _NS), _f32).at[0].set(g).at[1].set(b)
    return _tc_call(
        _k_out, grid,
        [_row_spec(_BRN, _NS), _row_spec(_BRN, 8), _full_spec((8, _NS)),
         _full_spec(w1.shape), _full_spec((8, w1.shape[1])),
         _full_spec(w2.shape), _full_spec((8, 8))],
        _row_spec(_BRN, 8), _sds((_NP, 8)),
    )(x, res, lnrows, w1, _brow(b1), w2, _brow(b2))


@jax.jit
def _run(u, v, boundary_norm, y_force, pos, edge_index, params):
    npad = _NP - _N

    def padn(a):
        return jnp.pad(a, ((0, npad),) + ((0, 0),) * (a.ndim - 1))

    row = edge_index[0].astype(_i32)
    col = edge_index[1].astype(_i32)
    fill = jnp.full((_EP - _E,), _N, _i32)
    rowp = jnp.concatenate([row, fill])
    colp = jnp.concatenate([col, fill])
    row2d = rowp.reshape(_EP // _CH, _CH)

    u_p = padn(u)
    vx = padn(v[:, :, 0])
    vy = padn(v[:, :, 1])
    bn_p = padn(boundary_norm)
    y_p = padn(y_force)
    ptab = jnp.pad(pos, ((0, npad), (0, 14)))
    res = (jnp.zeros((_NP, 8), _f32)
           .at[:_N, 0].set(u[:, -1])
           .at[:_N, 1:3].set(v[:, -1, :]))

    (w0, b0), (w1, b1) = params["emb"]
    x = _emb_call(u_p, vx, vy, bn_p, y_p, w0, b0, w1, b1)

    pr, pc = _make_gather(16)(ptab, ptab, rowp, colp)
    demb = _bessel_call(pr, pc)

    for blk in params["blocks"]:
        (we1, be1), (we2, be2) = blk["edge"]
        (wn1, bn1), (wn2, bn2) = blk["node"]
        wa = we1[:_NS]
        wb = we1[_NS:2 * _NS]
        wc = we1[2 * _NS:]
        a_tab, b_tab = _ab_call(x, wa, wb, be1)
        gr, gc = _make_gather(32)(a_tab, b_tab, rowp, colp)
        m = _edge_call(gr, gc, demb, wc, we2, be2)
        acc = _make_scatter()(m, row2d)
        x = _node_call(x, acc, wn1[:_NS], wn1[_NS:], bn1, wn2, bn2)

    g, b = params["ln"]
    (wr1, br1), (wr2, br2) = params["out_rot"]
    (ws1, bs1), (ws2, bs2) = params["out_scalar"]
    wc1 = jnp.concatenate([wr1, ws1], axis=1)
    bc1 = jnp.concatenate([br1, bs1])
    wc2 = (jnp.zeros((192, 8), _f32)
           .at[0:96, 1:3].set(wr2)
           .at[96:192, 0:1].set(ws2))
    bc2 = (jnp.zeros((8,), _f32)
           .at[0].set(bs2[0])
           .at[1:3].set(br2))
    o = _out_call(x, res, g, b, wc1, bc1, wc2, bc2)

    return o[:_N, 0], o[:_N, 1:3]


def kernel(u, v, boundary_norm, is_boundary, y_force, pos, edge_index, params):
    return _run(u, v, boundary_norm, y_force, pos, edge_index, params)


# speedup vs baseline: 2.4882x; 1.0806x over previous
import functools

import jax
import jax.numpy as jnp
from jax import lax
from jax.experimental import pallas as pl
from jax.experimental.pallas import tpu as pltpu
import jax.experimental.pallas.tpu_sc as plsc

_N = 50000
_E = 800000
_T = 10
_RDIM = 16
_NS = 32
_L = 5

_NP = 50176
_EP = 819200
_NWORK = 32
_PW = _EP // _NWORK
_CH = 128
_IB = 1024
_SUBROWS = _NP // 16

_BRN = 6272
_BRE = 8192

_f32 = jnp.float32
_i32 = jnp.int32


def _sds(shape):
    return jax.ShapeDtypeStruct(shape, _f32)


def _k_emb(u_ref, vx_ref, vy_ref, bn_ref, y_ref, w0_ref, b0_ref, w1_ref,
           b1_ref, o_ref):
    u = u_ref[...]
    vx = vx_ref[...]
    vy = vy_ref[...]
    bn = bn_ref[...]
    y = y_ref[...]
    vn = jnp.sqrt(vx * vx + vy * vy)
    bnx = bn[:, 0:1]
    bny = bn[:, 1:2]
    vd = vx * bnx + vy * bny
    bnn = jnp.sqrt(bnx * bnx + bny * bny)
    inv = jnp.concatenate([u, vn, vd, bnn, y], axis=1)
    h = jnp.dot(inv, w0_ref[...], preferred_element_type=_f32) + b0_ref[0:1, :]
    h = jnp.maximum(h, 0.0)
    o_ref[...] = jnp.dot(h, w1_ref[...], preferred_element_type=_f32) + b1_ref[0:1, :]


def _k_bessel(pr_ref, pc_ref, o_ref):
    dx = pr_ref[:, 0:1] - pc_ref[:, 0:1]
    dy = pr_ref[:, 1:2] - pc_ref[:, 1:2]
    d = jnp.sqrt(dx * dx + dy * dy)
    n = lax.broadcasted_iota(_i32, (1, _RDIM), 1).astype(_f32) + 1.0
    o_ref[...] = jnp.sqrt(2.0) * jnp.sin(n * jnp.pi * d) / (d + 1e-8)


def _k_ab(x_ref, wa_ref, wb_ref, b_ref, a_ref, bo_ref):
    x = x_ref[...]
    a_ref[...] = jnp.dot(x, wa_ref[...], preferred_element_type=_f32) + b_ref[0:1, :]
    bo_ref[...] = jnp.dot(x, wb_ref[...], preferred_element_type=_f32)


def _k_edge(gr_ref, gc_ref, de_ref, wc_ref, w2_ref, b2_ref, o_ref):
    pre = gr_ref[...] + gc_ref[...] + jnp.dot(
        de_ref[...], wc_ref[...], preferred_element_type=_f32)
    h = jnp.maximum(pre, 0.0)
    o_ref[...] = jnp.dot(h, w2_ref[...], preferred_element_type=_f32) + b2_ref[0:1, :]


def _k_node(x_ref, acc_ref, va_ref, vb_ref, c_ref, v2_ref, o_ref):
    x = x_ref[...]
    agg = acc_ref[0] + acc_ref[1]
    h = (jnp.dot(x, va_ref[...], preferred_element_type=_f32)
         + jnp.dot(agg, vb_ref[...], preferred_element_type=_f32)
         + c_ref[0:1, :])
    h = jnp.maximum(h, 0.0)
    o_ref[...] = x + jnp.dot(h, v2_ref[...], preferred_element_type=_f32) + c_ref[1:2, :]


def _k_out(x_ref, res_ref, ln_ref, w1_ref, b1_ref, w2_ref, b2_ref, o_ref):
    x = x_ref[...]
    mu = jnp.mean(x, axis=1, keepdims=True)
    xc = x - mu
    var = jnp.mean(xc * xc, axis=1, keepdims=True)
    xn = xc * jax.lax.rsqrt(var + 1e-5) * ln_ref[0:1, :] + ln_ref[1:2, :]
    h = jnp.dot(xn, w1_ref[...], preferred_element_type=_f32) + b1_ref[0:1, :]
    h = jnp.maximum(h, 0.0)
    o = jnp.dot(h, w2_ref[...], preferred_element_type=_f32) + b2_ref[0:1, :]
    o_ref[...] = o + res_ref[...]


def _tc_call(body, grid, in_specs, out_specs, out_shape):
    return pl.pallas_call(body, grid=grid, in_specs=in_specs,
                          out_specs=out_specs, out_shape=out_shape)


def _row_spec(br, w):
    return pl.BlockSpec((br, w), lambda i: (i, 0))


def _full_spec(shape):
    return pl.BlockSpec(shape, lambda i: tuple(0 for _ in shape))


@functools.cache
def _mesh():
    return plsc.VectorSubcoreMesh(core_axis_name="c", subcore_axis_name="s")


@functools.cache
def _make_gather(width):
    nchk = _PW // _CH
    nb = 4
    k = 2

    def body(tab_a, tab_b, idx_a, idx_b, out_a, out_b, ia, ib, bufa, bufb,
             gsem, wsem):
        wid = lax.axis_index("s") * 2 + lax.axis_index("c")
        base = wid * _PW
        pltpu.sync_copy(idx_a.at[pl.ds(base, _PW)], ia)
        pltpu.sync_copy(idx_b.at[pl.ds(base, _PW)], ib)

        def issue_gather(j, b):
            pltpu.async_copy(tab_a.at[ia.at[pl.ds(j * _CH, _CH)]],
                             bufa.at[b], gsem.at[b])
            pltpu.async_copy(tab_b.at[ib.at[pl.ds(j * _CH, _CH)]],
                             bufb.at[b], gsem.at[b])

        def wait_gather(b):
            pltpu.make_async_copy(out_a.at[pl.ds(0, _CH)], bufa.at[b],
                                  gsem.at[b]).wait()
            pltpu.make_async_copy(out_a.at[pl.ds(0, _CH)], bufb.at[b],
                                  gsem.at[b]).wait()

        def issue_write(j, b):
            pltpu.async_copy(bufa.at[b], out_a.at[pl.ds(base + j * _CH, _CH)],
                             wsem.at[b])
            pltpu.async_copy(bufb.at[b], out_b.at[pl.ds(base + j * _CH, _CH)],
                             wsem.at[b])

        def wait_write(b):
            pltpu.make_async_copy(out_a.at[pl.ds(0, _CH)], bufa.at[b],
                                  wsem.at[b]).wait()
            pltpu.make_async_copy(out_a.at[pl.ds(0, _CH)], bufb.at[b],
                                  wsem.at[b]).wait()

        for j in range(k):
            issue_gather(j, j)

        def superstep(s, carry):
            for b in range(nb):
                jv = s * nb + b
                b2 = (b + k) % nb

                @pl.when(jv + k < nchk)
                def _():
                    @pl.when(jv + k >= nb)
                    def _():
                        wait_write(b2)

                    issue_gather(jv + k, b2)

                wait_gather(b)
                issue_write(jv, b)
            return carry

        lax.fori_loop(0, nchk // nb, superstep, 0)
        for b in range(nb):
            wait_write(b)

    return pl.kernel(
        body,
        out_type=(_sds((_EP, width)), _sds((_EP, width))),
        mesh=_mesh(),
        scratch_types=[
            pltpu.VMEM((_PW,), _i32),
            pltpu.VMEM((_PW,), _i32),
            pltpu.VMEM((nb, _CH, width), _f32),
            pltpu.VMEM((nb, _CH, width), _f32),
            pltpu.SemaphoreType.DMA((nb,)),
            pltpu.SemaphoreType.DMA((nb,)),
        ],
        compiler_params=pltpu.CompilerParams(use_tc_tiling_on_sc=False),
    )


@functools.cache
def _make_scatter():
    nchk = _PW // _CH
    nb = 4
    k = 2
    zr = 392

    def body(m_hbm, row2d, out_hbm, ibufs, mbufs, zbuf, acc_sh, msem, ssem):
        cid = lax.axis_index("c")
        sid = lax.axis_index("s")
        wid = sid * 2 + cid
        z = jnp.zeros((16,), _f32)

        def zb(r, carry):
            zbuf[r, pl.ds(0, 16)] = z
            zbuf[r, pl.ds(16, 16)] = z
            return carry

        lax.fori_loop(0, zr, zb, 0)

        def zs(i, carry):
            pltpu.sync_copy(zbuf, acc_sh.at[pl.ds(sid * _SUBROWS + i * zr, zr)])
            return carry

        lax.fori_loop(0, _SUBROWS // zr, zs, 0)
        plsc.subcore_barrier()

        cbase = wid * nchk

        def issue_load(j, b):
            pltpu.async_copy(row2d.at[cbase + j], ibufs.at[b], msem.at[b])
            pltpu.async_copy(m_hbm.at[pl.ds((cbase + j) * _CH, _CH)],
                             mbufs.at[b], msem.at[b])

        def wait_load(b):
            pltpu.make_async_copy(m_hbm.at[pl.ds(0, _CH)], mbufs.at[b],
                                  msem.at[b]).wait()
            pltpu.make_async_copy(row2d.at[0], ibufs.at[b], msem.at[b]).wait()

        def issue_scat(b):
            pltpu.async_copy(mbufs.at[b], acc_sh.at[ibufs.at[b]], ssem.at[b],
                             add=True)

        def wait_scat(b):
            pltpu.make_async_copy(m_hbm.at[pl.ds(0, _CH)], mbufs.at[b],
                                  ssem.at[b]).wait()

        for j in range(k):
            issue_load(j, j)

        def superstep(s, carry):
            for b in range(nb):
                jv = s * nb + b
                b2 = (b + k) % nb

                @pl.when(jv + k < nchk)
                def _():
                    @pl.when(jv + k >= nb)
                    def _():
                        wait_scat(b2)

                    issue_load(jv + k, b2)

                wait_load(b)
                issue_scat(b)
            return carry

        lax.fori_loop(0, nchk // nb, superstep, 0)
        for b in range(nb):
            wait_scat(b)
        plsc.subcore_barrier()

        def dump(i, carry):
            r0 = sid * _SUBROWS + i * zr
            pltpu.sync_copy(acc_sh.at[pl.ds(r0, zr)], zbuf)
            pltpu.sync_copy(zbuf, out_hbm.at[cid, pl.ds(r0, zr)])
            return carry

        lax.fori_loop(0, _SUBROWS // zr, dump, 0)

    return pl.kernel(
        body,
        out_type=_sds((2, _NP, 32)),
        mesh=_mesh(),
        scratch_types=[
            pltpu.VMEM((nb, _CH), _i32),
            pltpu.VMEM((nb, _CH, 32), _f32),
            pltpu.VMEM((zr, 32), _f32),
            pltpu.VMEM_SHARED((_NP, 32), _f32),
            pltpu.SemaphoreType.DMA((nb,)),
            pltpu.SemaphoreType.DMA((nb,)),
        ],
        compiler_params=pltpu.CompilerParams(use_tc_tiling_on_sc=False),
    )


def _brow(vec, rows=8):
    return jnp.zeros((rows, vec.shape[0]), _f32).at[0].set(vec)


def _emb_call(u_p, vx, vy, bn_p, y_p, w0, b0, w1, b1):
    grid = (_NP // _BRN,)
    return _tc_call(
        _k_emb, grid,
        [
            _row_spec(_BRN, _T), _row_spec(_BRN, _T), _row_spec(_BRN, _T),
            _row_spec(_BRN, 2), _row_spec(_BRN, 1),
            _full_spec(w0.shape), _full_spec((8, 2 * _NS)),
            _full_spec(w1.shape), _full_spec((8, _NS)),
        ],
        _row_spec(_BRN, _NS), _sds((_NP, _NS)),
    )(u_p, vx, vy, bn_p, y_p, w0, _brow(b0), w1, _brow(b1))


def _bessel_call(pr, pc):
    grid = (_EP // _BRE,)
    return _tc_call(
        _k_bessel, grid,
        [_row_spec(_BRE, 16), _row_spec(_BRE, 16)],
        _row_spec(_BRE, _RDIM), _sds((_EP, _RDIM)),
    )(pr, pc)


def _ab_call(x, wa, wb, b1):
    grid = (_NP // _BRN,)
    return _tc_call(
        _k_ab, grid,
        [_row_spec(_BRN, _NS), _full_spec(wa.shape), _full_spec(wb.shape),
         _full_spec((8, _NS))],
        (_row_spec(_BRN, _NS), _row_spec(_BRN, _NS)),
        (_sds((_NP, _NS)), _sds((_NP, _NS))),
    )(x, wa, wb, _brow(b1))


def _edge_call(gr, gc, demb, wc, w2, b2):
    grid = (_EP // _BRE,)
    return _tc_call(
        _k_edge, grid,
        [_row_spec(_BRE, _NS), _row_spec(_BRE, _NS), _row_spec(_BRE, _RDIM),
         _full_spec(wc.shape), _full_spec(w2.shape), _full_spec((8, _NS))],
        _row_spec(_BRE, _NS), _sds((_EP, _NS)),
    )(gr, gc, demb, wc, w2, _brow(b2))


def _node_call(x, acc, va, vb, c1, v2, c2):
    grid = (_NP // _BRN,)
    crows = jnp.zeros((8, _NS), _f32).at[0].set(c1).at[1].set(c2)
    return _tc_call(
        _k_node, grid,
        [_row_spec(_BRN, _NS),
         pl.BlockSpec((2, _BRN, _NS), lambda i: (0, i, 0)),
         _full_spec(va.shape), _full_spec(vb.shape), _full_spec((8, _NS)),
         _full_spec(v2.shape)],
        _row_spec(_BRN, _NS), _sds((_NP, _NS)),
    )(x, acc, va, vb, crows, v2)


def _out_call(x, res, g, b, w1, b1, w2, b2):
    grid = (_NP // _BRN,)
    lnrows = jnp.zeros((8, _NS), _f32).at[0].set(g).at[1].set(b)
    return _tc_call(
        _k_out, grid,
        [_row_spec(_BRN, _NS), _row_spec(_BRN, 8), _full_spec((8, _NS)),
         _full_spec(w1.shape), _full_spec((8, w1.shape[1])),
         _full_spec(w2.shape), _full_spec((8, 8))],
        _row_spec(_BRN, 8), _sds((_NP, 8)),
    )(x, res, lnrows, w1, _brow(b1), w2, _brow(b2))


@jax.jit
def _run(u, v, boundary_norm, y_force, pos, edge_index, params):
    npad = _NP - _N

    def padn(a):
        return jnp.pad(a, ((0, npad),) + ((0, 0),) * (a.ndim - 1))

    row = edge_index[0].astype(_i32)
    col = edge_index[1].astype(_i32)
    fill = jnp.full((_EP - _E,), _N, _i32)
    rowp = jnp.concatenate([row, fill])
    colp = jnp.concatenate([col, fill])
    row2d = rowp.reshape(_EP // _CH, _CH)

    u_p = padn(u)
    vx = padn(v[:, :, 0])
    vy = padn(v[:, :, 1])
    bn_p = padn(boundary_norm)
    y_p = padn(y_force)
    ptab = jnp.pad(pos, ((0, npad), (0, 14)))
    res = (jnp.zeros((_NP, 8), _f32)
           .at[:_N, 0].set(u[:, -1])
           .at[:_N, 1:3].set(v[:, -1, :]))

    (w0, b0), (w1, b1) = params["emb"]
    x = _emb_call(u_p, vx, vy, bn_p, y_p, w0, b0, w1, b1)

    pr, pc = _make_gather(16)(ptab, ptab, rowp, colp)
    demb = _bessel_call(pr, pc)

    for blk in params["blocks"]:
        (we1, be1), (we2, be2) = blk["edge"]
        (wn1, bn1), (wn2, bn2) = blk["node"]
        wa = we1[:_NS]
        wb = we1[_NS:2 * _NS]
        wc = we1[2 * _NS:]
        a_tab, b_tab = _ab_call(x, wa, wb, be1)
        gr, gc = _make_gather(32)(a_tab, b_tab, rowp, colp)
        m = _edge_call(gr, gc, demb, wc, we2, be2)
        acc = _make_scatter()(m, row2d)
        x = _node_call(x, acc, wn1[:_NS], wn1[_NS:], bn1, wn2, bn2)

    g, b = params["ln"]
    (wr1, br1), (wr2, br2) = params["out_rot"]
    (ws1, bs1), (ws2, bs2) = params["out_scalar"]
    wc1 = jnp.concatenate([wr1, ws1], axis=1)
    bc1 = jnp.concatenate([br1, bs1])
    wc2 = (jnp.zeros((192, 8), _f32)
           .at[0:96, 1:3].set(wr2)
           .at[96:192, 0:1].set(ws2))
    bc2 = (jnp.zeros((8,), _f32)
           .at[0].set(bs2[0])
           .at[1:3].set(br2))
    o = _out_call(x, res, g, b, wc1, bc1, wc2, bc2)

    return o[:_N, 0], o[:_N, 1:3]


def kernel(u, v, boundary_norm, is_boundary, y_force, pos, edge_index, params):
    return _run(u, v, boundary_norm, y_force, pos, edge_index, params)


# speedup vs baseline: 6.9232x; 2.7824x over previous
import functools

import jax
import jax.numpy as jnp
from jax import lax
from jax.experimental import pallas as pl
from jax.experimental.pallas import tpu as pltpu
import jax.experimental.pallas.tpu_sc as plsc

_N = 50000
_E = 800000
_T = 10
_RDIM = 16
_NS = 32
_L = 5

_NP = 50176
_EP = 819200
_NWORK = 32
_PW = _EP // _NWORK
_CH = 128
_IB = 1024
_SUBROWS = _NP // 16

_BRN = 6272
_BRN4 = 1568
_BRE4 = 2048
_BRD = 1024

_f32 = jnp.float32
_i32 = jnp.int32


def _sds(shape):
    return jax.ShapeDtypeStruct(shape, _f32)


def _k_emb(u_ref, vx_ref, vy_ref, bn_ref, y_ref, w0_ref, b0_ref, w1_ref,
           b1_ref, o_ref):
    u = u_ref[...]
    vx = vx_ref[...]
    vy = vy_ref[...]
    bn = bn_ref[...]
    y = y_ref[...]
    vn = jnp.sqrt(vx * vx + vy * vy)
    bnx = bn[:, 0:1]
    bny = bn[:, 1:2]
    vd = vx * bnx + vy * bny
    bnn = jnp.sqrt(bnx * bnx + bny * bny)
    inv = jnp.concatenate([u, vn, vd, bnn, y], axis=1)
    h = jnp.dot(inv, w0_ref[...], preferred_element_type=_f32) + b0_ref[0:1, :]
    h = jnp.maximum(h, 0.0)
    o_ref[...] = jnp.dot(h, w1_ref[...], preferred_element_type=_f32) + b1_ref[0:1, :]


def _k_bessel(pr_ref, pc_ref, sb_ref, nv_ref, o_ref):
    diff = pr_ref[...] - pc_ref[...]
    w = diff * diff
    d = jnp.sqrt(jnp.dot(w, sb_ref[...], preferred_element_type=_f32))
    nv = nv_ref[0:1, :]
    o_ref[...] = jnp.sqrt(2.0) * jnp.sin(nv * jnp.pi * d) / (d + 1e-8)


def _k_ab(x_ref, wa_ref, wb_ref, b_ref, a_ref, bo_ref):
    x = x_ref[...]
    a_ref[...] = jnp.dot(x, wa_ref[...], preferred_element_type=_f32) + b_ref[0:1, :]
    bo_ref[...] = jnp.dot(x, wb_ref[...], preferred_element_type=_f32)


def _k_edge(gr_ref, gc_ref, de_ref, wc_ref, w2_ref, b2_ref, o_ref):
    pre = gr_ref[...] + gc_ref[...] + jnp.dot(
        de_ref[...], wc_ref[...], preferred_element_type=_f32)
    h = jnp.maximum(pre, 0.0)
    o_ref[...] = jnp.dot(h, w2_ref[...], preferred_element_type=_f32) + b2_ref[0:1, :]


def _k_node(x_ref, acc_ref, va_ref, vb_ref, c_ref, v2_ref, o_ref):
    x = x_ref[...]
    agg = acc_ref[0] + acc_ref[1]
    h = (jnp.dot(x, va_ref[...], preferred_element_type=_f32)
         + jnp.dot(agg, vb_ref[...], preferred_element_type=_f32)
         + c_ref[0:1, :])
    h = jnp.maximum(h, 0.0)
    o_ref[...] = x + jnp.dot(h, v2_ref[...], preferred_element_type=_f32) + c_ref[1:2, :]


def _k_out(x_ref, res_ref, ln_ref, w1_ref, b1_ref, w2_ref, b2_ref, o_ref):
    x = x_ref[...]
    mu = jnp.mean(x, axis=1, keepdims=True)
    xc = x - mu
    var = jnp.mean(xc * xc, axis=1, keepdims=True)
    xn = xc * jax.lax.rsqrt(var + 1e-5) * ln_ref[0:1, :] + ln_ref[1:2, :]
    h = jnp.dot(xn, w1_ref[...], preferred_element_type=_f32) + b1_ref[0:1, :]
    h = jnp.maximum(h, 0.0)
    o = jnp.dot(h, w2_ref[...], preferred_element_type=_f32) + b2_ref[0:1, :]
    o_ref[...] = o + res_ref[...]


def _tc_call(body, grid, in_specs, out_specs, out_shape):
    return pl.pallas_call(body, grid=grid, in_specs=in_specs,
                          out_specs=out_specs, out_shape=out_shape)


def _row_spec(br, w):
    return pl.BlockSpec((br, w), lambda i: (i, 0))


def _full_spec(shape):
    return pl.BlockSpec(shape, lambda i: tuple(0 for _ in shape))


@functools.cache
def _mesh():
    return plsc.VectorSubcoreMesh(core_axis_name="c", subcore_axis_name="s")


@functools.cache
def _make_gather(width):
    nchk = _PW // _CH
    nb = 4
    k = 2

    def body(tab_a, tab_b, idx_a, idx_b, out_a, out_b, ia, ib, bufa, bufb,
             gsem, wsem):
        wid = lax.axis_index("s") * 2 + lax.axis_index("c")
        base = wid * _PW
        pltpu.sync_copy(idx_a.at[pl.ds(base, _PW)], ia)
        pltpu.sync_copy(idx_b.at[pl.ds(base, _PW)], ib)

        def issue_gather(j, b):
            pltpu.async_copy(tab_a.at[ia.at[pl.ds(j * _CH, _CH)]],
                             bufa.at[b], gsem.at[b])
            pltpu.async_copy(tab_b.at[ib.at[pl.ds(j * _CH, _CH)]],
                             bufb.at[b], gsem.at[b])

        def wait_gather(b):
            pltpu.make_async_copy(out_a.at[pl.ds(0, _CH)], bufa.at[b],
                                  gsem.at[b]).wait()
            pltpu.make_async_copy(out_a.at[pl.ds(0, _CH)], bufb.at[b],
                                  gsem.at[b]).wait()

        def issue_write(j, b):
            pltpu.async_copy(bufa.at[b], out_a.at[pl.ds(base + j * _CH, _CH)],
                             wsem.at[b])
            pltpu.async_copy(bufb.at[b], out_b.at[pl.ds(base + j * _CH, _CH)],
                             wsem.at[b])

        def wait_write(b):
            pltpu.make_async_copy(out_a.at[pl.ds(0, _CH)], bufa.at[b],
                                  wsem.at[b]).wait()
            pltpu.make_async_copy(out_a.at[pl.ds(0, _CH)], bufb.at[b],
                                  wsem.at[b]).wait()

        for j in range(k):
            issue_gather(j, j)

        def superstep(s, carry):
            for b in range(nb):
                jv = s * nb + b
                b2 = (b + k) % nb

                @pl.when(jv + k < nchk)
                def _():
                    @pl.when(jv + k >= nb)
                    def _():
                        wait_write(b2)

                    issue_gather(jv + k, b2)

                wait_gather(b)
                issue_write(jv, b)
            return carry

        lax.fori_loop(0, nchk // nb, superstep, 0)
        for b in range(nb):
            wait_write(b)

    return pl.kernel(
        body,
        out_type=(_sds((_EP, width)), _sds((_EP, width))),
        mesh=_mesh(),
        scratch_types=[
            pltpu.VMEM((_PW,), _i32),
            pltpu.VMEM((_PW,), _i32),
            pltpu.VMEM((nb, _CH, width), _f32),
            pltpu.VMEM((nb, _CH, width), _f32),
            pltpu.SemaphoreType.DMA((nb,)),
            pltpu.SemaphoreType.DMA((nb,)),
        ],
        compiler_params=pltpu.CompilerParams(use_tc_tiling_on_sc=False),
    )


@functools.cache
def _make_scatter():
    nchk = _PW // _CH
    nb = 4
    k = 2
    zr = 392

    def body(m_hbm, row2d, out_hbm, ibufs, mbufs, zbuf, acc_sh, msem, ssem):
        cid = lax.axis_index("c")
        sid = lax.axis_index("s")
        wid = sid * 2 + cid
        z = jnp.zeros((16,), _f32)

        def zb(r, carry):
            zbuf[r, pl.ds(0, 16)] = z
            zbuf[r, pl.ds(16, 16)] = z
            return carry

        lax.fori_loop(0, zr, zb, 0)

        def zs(i, carry):
            pltpu.sync_copy(zbuf, acc_sh.at[pl.ds(sid * _SUBROWS + i * zr, zr)])
            return carry

        lax.fori_loop(0, _SUBROWS // zr, zs, 0)
        plsc.subcore_barrier()

        cbase = wid * nchk

        def issue_load(j, b):
            pltpu.async_copy(row2d.at[cbase + j], ibufs.at[b], msem.at[b])
            pltpu.async_copy(m_hbm.at[pl.ds((cbase + j) * _CH, _CH)],
                             mbufs.at[b], msem.at[b])

        def wait_load(b):
            pltpu.make_async_copy(m_hbm.at[pl.ds(0, _CH)], mbufs.at[b],
                                  msem.at[b]).wait()
            pltpu.make_async_copy(row2d.at[0], ibufs.at[b], msem.at[b]).wait()

        def issue_scat(b):
            pltpu.async_copy(mbufs.at[b], acc_sh.at[ibufs.at[b]], ssem.at[b],
                             add=True)

        def wait_scat(b):
            pltpu.make_async_copy(m_hbm.at[pl.ds(0, _CH)], mbufs.at[b],
                                  ssem.at[b]).wait()

        for j in range(k):
            issue_load(j, j)

        def superstep(s, carry):
            for b in range(nb):
                jv = s * nb + b
                b2 = (b + k) % nb

                @pl.when(jv + k < nchk)
                def _():
                    @pl.when(jv + k >= nb)
                    def _():
                        wait_scat(b2)

                    issue_load(jv + k, b2)

                wait_load(b)
                issue_scat(b)
            return carry

        lax.fori_loop(0, nchk // nb, superstep, 0)
        for b in range(nb):
            wait_scat(b)
        plsc.subcore_barrier()

        def dump(i, carry):
            r0 = sid * _SUBROWS + i * zr
            pltpu.sync_copy(acc_sh.at[pl.ds(r0, zr)], zbuf)
            pltpu.sync_copy(zbuf, out_hbm.at[cid, pl.ds(r0, zr)])
            return carry

        lax.fori_loop(0, _SUBROWS // zr, dump, 0)

    return pl.kernel(
        body,
        out_type=_sds((2, _NP, 32)),
        mesh=_mesh(),
        scratch_types=[
            pltpu.VMEM((nb, _CH), _i32),
            pltpu.VMEM((nb, _CH, 32), _f32),
            pltpu.VMEM((zr, 32), _f32),
            pltpu.VMEM_SHARED((_NP, 32), _f32),
            pltpu.SemaphoreType.DMA((nb,)),
            pltpu.SemaphoreType.DMA((nb,)),
        ],
        compiler_params=pltpu.CompilerParams(use_tc_tiling_on_sc=False),
    )


def _brow(vec, rows=8):
    return jnp.zeros((rows, vec.shape[0]), _f32).at[0].set(vec)


def _bd4(w):
    a, b = w.shape
    z = jnp.zeros((4 * a, 4 * b), _f32)
    for i in range(4):
        z = z.at[i * a:(i + 1) * a, i * b:(i + 1) * b].set(w)
    return z


def _tile4(vec):
    return jnp.tile(vec, 4)


def _emb_call(u_p, vx, vy, bn_p, y_p, w0, b0, w1, b1):
    grid = (_NP // _BRN,)
    return _tc_call(
        _k_emb, grid,
        [
            _row_spec(_BRN, _T), _row_spec(_BRN, _T), _row_spec(_BRN, _T),
            _row_spec(_BRN, 2), _row_spec(_BRN, 1),
            _full_spec(w0.shape), _full_spec((8, 2 * _NS)),
            _full_spec(w1.shape), _full_spec((8, _NS)),
        ],
        _row_spec(_BRN, _NS), _sds((_NP, _NS)),
    )(u_p, vx, vy, bn_p, y_p, w0, _brow(b0), w1, _brow(b1))


def _bessel_call(pr, pc):
    ii = jnp.arange(128)
    sb = ((ii[:, None] // 16 == ii[None, :] // 16)
          & (ii[:, None] % 16 < 2)).astype(_f32)
    nv = _brow(jnp.tile(jnp.arange(1, 17, dtype=_f32), 8))
    rows = _EP // 8
    grid = (rows // _BRD,)
    out = _tc_call(
        _k_bessel, grid,
        [_row_spec(_BRD, 128), _row_spec(_BRD, 128), _full_spec((128, 128)),
         _full_spec((8, 128))],
        _row_spec(_BRD, 128), _sds((rows, 128)),
    )(pr.reshape(rows, 128), pc.reshape(rows, 128), sb, nv)
    return out.reshape(_EP // 4, 64)


def _ab_call(x4, wa, wb, b1):
    grid = (_NP // 4 // _BRN4,)
    return _tc_call(
        _k_ab, grid,
        [_row_spec(_BRN4, 128), _full_spec((128, 128)), _full_spec((128, 128)),
         _full_spec((8, 128))],
        (_row_spec(_BRN4, 128), _row_spec(_BRN4, 128)),
        (_sds((_NP // 4, 128)), _sds((_NP // 4, 128))),
    )(x4, _bd4(wa), _bd4(wb), _brow(_tile4(b1)))


def _edge_call(gr, gc, demb4, wc, w2, b2):
    rows = _EP // 4
    grid = (rows // _BRE4,)
    return _tc_call(
        _k_edge, grid,
        [_row_spec(_BRE4, 128), _row_spec(_BRE4, 128), _row_spec(_BRE4, 64),
         _full_spec((64, 128)), _full_spec((128, 128)), _full_spec((8, 128))],
        _row_spec(_BRE4, 128), _sds((rows, 128)),
    )(gr.reshape(rows, 128), gc.reshape(rows, 128), demb4,
      _bd4(wc), _bd4(w2), _brow(_tile4(b2)))


def _node_call(x4, acc, va, vb, c1, v2, c2):
    grid = (_NP // 4 // _BRN4,)
    crows = (jnp.zeros((8, 128), _f32)
             .at[0].set(_tile4(c1))
             .at[1].set(_tile4(c2)))
    return _tc_call(
        _k_node, grid,
        [_row_spec(_BRN4, 128),
         pl.BlockSpec((2, _BRN4, 128), lambda i: (0, i, 0)),
         _full_spec((128, 128)), _full_spec((128, 128)), _full_spec((8, 128)),
         _full_spec((128, 128))],
        _row_spec(_BRN4, 128), _sds((_NP // 4, 128)),
    )(x4, acc.reshape(2, _NP // 4, 128), _bd4(va), _bd4(vb), crows, _bd4(v2))


def _out_call(x, res, g, b, w1, b1, w2, b2):
    grid = (_NP // _BRN,)
    lnrows = jnp.zeros((8, _NS), _f32).at[0].set(g).at[1].set(b)
    return _tc_call(
        _k_out, grid,
        [_row_spec(_BRN, _NS), _row_spec(_BRN, 8), _full_spec((8, _NS)),
         _full_spec(w1.shape), _full_spec((8, w1.shape[1])),
         _full_spec(w2.shape), _full_spec((8, 8))],
        _row_spec(_BRN, 8), _sds((_NP, 8)),
    )(x, res, lnrows, w1, _brow(b1), w2, _brow(b2))


@jax.jit
def _run(u, v, boundary_norm, y_force, pos, edge_index, params):
    npad = _NP - _N

    def padn(a):
        return jnp.pad(a, ((0, npad),) + ((0, 0),) * (a.ndim - 1))

    row = edge_index[0].astype(_i32)
    col = edge_index[1].astype(_i32)
    fill = jnp.full((_EP - _E,), _N, _i32)
    rowp = jnp.concatenate([row, fill])
    colp = jnp.concatenate([col, fill])
    row2d = rowp.reshape(_EP // _CH, _CH)

    u_p = padn(u)
    vx = padn(v[:, :, 0])
    vy = padn(v[:, :, 1])
    bn_p = padn(boundary_norm)
    y_p = padn(y_force)
    ptab = jnp.pad(pos, ((0, npad), (0, 14)))
    res = (jnp.zeros((_NP, 8), _f32)
           .at[:_N, 0].set(u[:, -1])
           .at[:_N, 1:3].set(v[:, -1, :]))

    (w0, b0), (w1, b1) = params["emb"]
    x = _emb_call(u_p, vx, vy, bn_p, y_p, w0, b0, w1, b1)
    x4 = x.reshape(_NP // 4, 128)

    pr, pc = _make_gather(16)(ptab, ptab, rowp, colp)
    demb4 = _bessel_call(pr, pc)

    for blk in params["blocks"]:
        (we1, be1), (we2, be2) = blk["edge"]
        (wn1, bn1), (wn2, bn2) = blk["node"]
        wa = we1[:_NS]
        wb = we1[_NS:2 * _NS]
        wc = we1[2 * _NS:]
        a4, b4 = _ab_call(x4, wa, wb, be1)
        gr, gc = _make_gather(32)(a4.reshape(_NP, _NS), b4.reshape(_NP, _NS),
                                  rowp, colp)
        m = _edge_call(gr, gc, demb4, wc, we2, be2)
        acc = _make_scatter()(m.reshape(_EP, _NS), row2d)
        x4 = _node_call(x4, acc, wn1[:_NS], wn1[_NS:], bn1, wn2, bn2)

    x = x4.reshape(_NP, _NS)
    g, b = params["ln"]
    (wr1, br1), (wr2, br2) = params["out_rot"]
    (ws1, bs1), (ws2, bs2) = params["out_scalar"]
    wc1 = jnp.concatenate([wr1, ws1], axis=1)
    bc1 = jnp.concatenate([br1, bs1])
    wc2 = (jnp.zeros((192, 8), _f32)
           .at[0:96, 1:3].set(wr2)
           .at[96:192, 0:1].set(ws2))
    bc2 = (jnp.zeros((8,), _f32)
           .at[0].set(bs2[0])
           .at[1:3].set(br2))
    o = _out_call(x, res, g, b, wc1, bc1, wc2, bc2)

    return o[:_N, 0], o[:_N, 1:3]


def kernel(u, v, boundary_norm, is_boundary, y_force, pos, edge_index, params):
    return _run(u, v, boundary_norm, y_force, pos, edge_index, params)
